# baseline (speedup 1.0000x reference)
import functools

import jax
import jax.numpy as jnp
from jax import lax
from jax.experimental import pallas as pl
from jax.experimental.pallas import tpu as pltpu
from jax.experimental.pallas import tpu_sc as plsc

N = 4096
D = 128
TC_R = 128
K = 3


def _tc_body(a_ref, bT_ref, sqr_ref, sqc_ref, bmat_ref, vals_ref, idx_ref):
    a = a_ref[...]
    bT = bT_ref[...]
    gram = jnp.dot(a, bT, preferred_element_type=jnp.float32)
    d2 = sqr_ref[...] + sqc_ref[...] - 2.0 * gram
    big = jnp.float32(jnp.inf)
    work = jnp.where(d2 > 0.0, d2, big)
    row0 = pl.program_id(0) * TC_R
    grows = row0 + lax.broadcasted_iota(jnp.int32, (TC_R, 1), 0)

    m1 = jnp.min(work, axis=1, keepdims=True)
    m2 = jnp.min(jnp.where(work > m1, work, big), axis=1, keepdims=True)
    m3 = jnp.min(jnp.where(work > m2, work, big), axis=1, keepdims=True)

    ohsum = jnp.where(work == m1, 1.0,
                      jnp.where(work == m2, 256.0,
                                jnp.where(work == m3, 65536.0, 0.0)))
    r = jnp.dot(ohsum, bmat_ref[...], preferred_element_type=jnp.float32)
    cnt = r[:, 2:3]
    unique = jnp.max(jnp.abs(cnt - 65793.0)) == 0.0

    @pl.when(unique)
    def _fast():
        d2top = jnp.concatenate([m1, m2, m3], axis=1)
        his = r[:, 0:1].astype(jnp.int32)
        los = r[:, 1:2].astype(jnp.int32)
        i1 = 64 * (his & 255) + (los & 255)
        i2 = 64 * ((his >> 8) & 255) + ((los >> 8) & 255)
        i3 = 64 * (his >> 16) + (los >> 16)
        idxs = jnp.concatenate([i1, i2, i3], axis=1)
        vals_ref[...] = 1.0 / (jnp.sqrt(d2top) + 1.0)
        idx_ref[...] = grows * N + idxs

    @pl.when(jnp.logical_not(unique))
    def _exact():
        cols = lax.broadcasted_iota(jnp.int32, (TC_R, N), 1)
        w = work
        vs, ids = [], []
        for _ in range(K):
            m = jnp.min(w, axis=1, keepdims=True)
            sel = w == m
            idx = jnp.min(jnp.where(sel, cols, N), axis=1, keepdims=True)
            vs.append(m)
            ids.append(grows * N + idx)
            w = jnp.where(cols == idx, big, w)
        d2top = jnp.concatenate(vs, axis=1)
        vals_ref[...] = 1.0 / (jnp.sqrt(d2top) + 1.0)
        idx_ref[...] = jnp.concatenate(ids, axis=1)


def _tc_top3(x, xT, sq_col, sq_row, bmat):
    return pl.pallas_call(
        _tc_body,
        grid=(N // TC_R,),
        in_specs=[
            pl.BlockSpec((TC_R, D), lambda i: (i, 0)),
            pl.BlockSpec((D, N), lambda i: (0, 0)),
            pl.BlockSpec((TC_R, 1), lambda i: (i, 0)),
            pl.BlockSpec((1, N), lambda i: (0, 0)),
            pl.BlockSpec((N, 128), lambda i: (0, 0)),
        ],
        out_specs=[
            pl.BlockSpec((TC_R, K), lambda i: (i, 0)),
            pl.BlockSpec((TC_R, K), lambda i: (i, 0)),
        ],
        out_shape=[
            jax.ShapeDtypeStruct((N, K), jnp.float32),
            jax.ShapeDtypeStruct((N, K), jnp.int32),
        ],
    )(x, xT, sq_col, sq_row, bmat)


def _sc_combine(idxT, valsT, memflat):
    info = plsc.get_sparse_core_info()
    nw = info.num_cores * info.num_subcores
    rpw = N // nw
    mesh = plsc.VectorSubcoreMesh(core_axis_name="c", subcore_axis_name="s")

    @functools.partial(
        pl.kernel,
        mesh=mesh,
        out_type=jax.ShapeDtypeStruct((N,), jnp.float32),
        scratch_types=[
            pltpu.VMEM((K, rpw), jnp.int32),
            pltpu.VMEM((K, rpw), jnp.int32),
            pltpu.VMEM((K, rpw), jnp.float32),
            pltpu.VMEM((rpw,), jnp.float32),
            pltpu.SemaphoreType.DMA,
        ],
    )
    def k(idx_hbm, vals_hbm, mem_hbm, out_hbm, idx_v, mem_v, vals_v, out_v, sem):
        wid = lax.axis_index("s") * info.num_cores + lax.axis_index("c")
        base = wid * rpw
        nl = info.num_lanes
        pltpu.sync_copy(idx_hbm.at[:, pl.ds(base, rpw)], idx_v)
        pltpu.sync_copy(vals_hbm.at[:, pl.ds(base, rpw)], vals_v)
        for kk in range(K):
            pltpu.async_copy(mem_hbm.at[idx_v.at[kk]], mem_v.at[kk], sem).wait()
        for j in range(rpw // nl):
            s = pl.ds(j * nl, nl)
            acc = vals_v[0, s] * mem_v[0, s].astype(jnp.float32)
            acc = acc + vals_v[1, s] * mem_v[1, s].astype(jnp.float32)
            acc = acc + vals_v[2, s] * mem_v[2, s].astype(jnp.float32)
            out_v[s] = acc / 3.0
        pltpu.sync_copy(out_v, out_hbm.at[pl.ds(base, rpw)])

    return k(idxT, valsT, memflat)


def kernel(items_embeddings, membership):
    x = items_embeddings
    sq = jnp.sum(x * x, axis=1)
    col = jnp.arange(N, dtype=jnp.int32)
    bmat = (jnp.zeros((N, 128), jnp.float32)
            .at[:, 0].set((col // 64).astype(jnp.float32))
            .at[:, 1].set((col % 64).astype(jnp.float32))
            .at[:, 2].set(1.0))
    vals, idx = _tc_top3(x, x.T, sq[:, None], sq[None, :], bmat)
    return _sc_combine(idx.T, vals.T, membership.reshape(-1))


# speedup vs baseline: 1.0442x; 1.0442x over previous
import functools

import jax
import jax.numpy as jnp
from jax import lax
from jax.experimental import pallas as pl
from jax.experimental.pallas import tpu as pltpu
from jax.experimental.pallas import tpu_sc as plsc

N = 4096
D = 128
TC_R = 128
K = 3


def _tc_body(a_ref, bT_ref, sqr_ref, sqc_ref, bmat_ref, vals_ref, idx_ref):
    a = a_ref[...]
    bT = bT_ref[...]
    gram = jnp.dot(a, bT, preferred_element_type=jnp.float32)
    d2 = sqr_ref[...] + sqc_ref[...] - 2.0 * gram
    big = jnp.float32(jnp.inf)
    work = jnp.where(d2 > 0.0, d2, big)
    row0 = pl.program_id(0) * TC_R
    grows = row0 + lax.broadcasted_iota(jnp.int32, (TC_R, 1), 0)

    m1 = jnp.min(work, axis=1, keepdims=True)
    m2 = jnp.min(jnp.where(work > m1, work, big), axis=1, keepdims=True)
    m3 = jnp.min(jnp.where(work > m2, work, big), axis=1, keepdims=True)

    ohsum = jnp.where(work == m1, 1.0,
                      jnp.where(work == m2, 256.0,
                                jnp.where(work == m3, 65536.0, 0.0)))
    r = jnp.dot(ohsum, bmat_ref[...], preferred_element_type=jnp.float32)
    cnt = r[:, 2:3]
    unique = jnp.max(jnp.abs(cnt - 65793.0)) == 0.0
    unique = jnp.logical_or(unique, True)

    @pl.when(unique)
    def _fast():
        d2top = jnp.concatenate([m1, m2, m3], axis=1)
        his = r[:, 0:1].astype(jnp.int32)
        los = r[:, 1:2].astype(jnp.int32)
        i1 = 64 * (his & 255) + (los & 255)
        i2 = 64 * ((his >> 8) & 255) + ((los >> 8) & 255)
        i3 = 64 * (his >> 16) + (los >> 16)
        idxs = jnp.concatenate([i1, i2, i3], axis=1)
        vals_ref[...] = 1.0 / (jnp.sqrt(d2top) + 1.0)
        idx_ref[...] = grows * N + idxs

    @pl.when(jnp.logical_not(unique))
    def _exact():
        cols = lax.broadcasted_iota(jnp.int32, (TC_R, N), 1)
        w = work
        vs, ids = [], []
        for _ in range(K):
            m = jnp.min(w, axis=1, keepdims=True)
            sel = w == m
            idx = jnp.min(jnp.where(sel, cols, N), axis=1, keepdims=True)
            vs.append(m)
            ids.append(grows * N + idx)
            w = jnp.where(cols == idx, big, w)
        d2top = jnp.concatenate(vs, axis=1)
        vals_ref[...] = 1.0 / (jnp.sqrt(d2top) + 1.0)
        idx_ref[...] = jnp.concatenate(ids, axis=1)


def _tc_top3(x, xT, sq_col, sq_row, bmat):
    return pl.pallas_call(
        _tc_body,
        grid=(N // TC_R,),
        in_specs=[
            pl.BlockSpec((TC_R, D), lambda i: (i, 0)),
            pl.BlockSpec((D, N), lambda i: (0, 0)),
            pl.BlockSpec((TC_R, 1), lambda i: (i, 0)),
            pl.BlockSpec((1, N), lambda i: (0, 0)),
            pl.BlockSpec((N, 128), lambda i: (0, 0)),
        ],
        out_specs=[
            pl.BlockSpec((TC_R, K), lambda i: (i, 0)),
            pl.BlockSpec((TC_R, K), lambda i: (i, 0)),
        ],
        out_shape=[
            jax.ShapeDtypeStruct((N, K), jnp.float32),
            jax.ShapeDtypeStruct((N, K), jnp.int32),
        ],
    )(x, xT, sq_col, sq_row, bmat)


def _sc_combine(idxT, valsT, memflat):
    info = plsc.get_sparse_core_info()
    nw = info.num_cores * info.num_subcores
    rpw = N // nw
    mesh = plsc.VectorSubcoreMesh(core_axis_name="c", subcore_axis_name="s")

    @functools.partial(
        pl.kernel,
        mesh=mesh,
        out_type=jax.ShapeDtypeStruct((N,), jnp.float32),
        scratch_types=[
            pltpu.VMEM((K, rpw), jnp.int32),
            pltpu.VMEM((K, rpw), jnp.int32),
            pltpu.VMEM((K, rpw), jnp.float32),
            pltpu.VMEM((rpw,), jnp.float32),
            pltpu.SemaphoreType.DMA,
        ],
    )
    def k(idx_hbm, vals_hbm, mem_hbm, out_hbm, idx_v, mem_v, vals_v, out_v, sem):
        wid = lax.axis_index("s") * info.num_cores + lax.axis_index("c")
        base = wid * rpw
        nl = info.num_lanes
        pltpu.sync_copy(idx_hbm.at[:, pl.ds(base, rpw)], idx_v)
        pltpu.sync_copy(vals_hbm.at[:, pl.ds(base, rpw)], vals_v)
        for kk in range(K):
            pltpu.async_copy(mem_hbm.at[idx_v.at[kk]], mem_v.at[kk], sem).wait()
        for j in range(rpw // nl):
            s = pl.ds(j * nl, nl)
            acc = vals_v[0, s] * mem_v[0, s].astype(jnp.float32)
            acc = acc + vals_v[1, s] * mem_v[1, s].astype(jnp.float32)
            acc = acc + vals_v[2, s] * mem_v[2, s].astype(jnp.float32)
            out_v[s] = acc / 3.0
        pltpu.sync_copy(out_v, out_hbm.at[pl.ds(base, rpw)])

    return k(idxT, valsT, memflat)


def kernel(items_embeddings, membership):
    x = items_embeddings
    sq = jnp.sum(x * x, axis=1)
    col = jnp.arange(N, dtype=jnp.int32)
    bmat = (jnp.zeros((N, 128), jnp.float32)
            .at[:, 0].set((col // 64).astype(jnp.float32))
            .at[:, 1].set((col % 64).astype(jnp.float32))
            .at[:, 2].set(1.0))
    vals, idx = _tc_top3(x, x.T, sq[:, None], sq[None, :], bmat)
    return _sc_combine(idx.T, vals.T, membership.reshape(-1))


# speedup vs baseline: 1.4810x; 1.4184x over previous
import functools

import jax
import jax.numpy as jnp
from jax import lax
from jax.experimental import pallas as pl
from jax.experimental.pallas import tpu as pltpu
from jax.experimental.pallas import tpu_sc as plsc

N = 4096
D = 128
TC_R = 256
K = 3


def _tc_body(a_ref, bT_ref, sqr_ref, sqc_ref, vals_ref, idx_ref):
    a = a_ref[...]
    bT = bT_ref[...]
    gram = jnp.dot(a, bT, preferred_element_type=jnp.float32)
    d2 = sqr_ref[...] + sqc_ref[...] - 2.0 * gram
    big = jnp.float32(jnp.inf)
    work = jnp.where(d2 > 0.0, d2, big)
    row0 = pl.program_id(0) * TC_R
    grows = row0 + lax.broadcasted_iota(jnp.int32, (TC_R, 1), 0)
    colf = lax.broadcasted_iota(jnp.int32, (TC_R, N), 1).astype(jnp.float32)
    bigc = jnp.float32(N)
    vs, ids = [], []
    for _ in range(K):
        m = jnp.min(work, axis=1, keepdims=True)
        sel = work == m
        idxf = jnp.min(jnp.where(sel, colf, bigc), axis=1, keepdims=True)
        vs.append(m)
        ids.append(grows * N + idxf.astype(jnp.int32))
        work = jnp.where(colf == idxf, big, work)
    d2top = jnp.concatenate(vs, axis=1)
    vals_ref[...] = 1.0 / (jnp.sqrt(d2top) + 1.0)
    idx_ref[...] = jnp.concatenate(ids, axis=1)


def _tc_top3(x, xT, sq_col, sq_row):
    return pl.pallas_call(
        _tc_body,
        grid=(N // TC_R,),
        in_specs=[
            pl.BlockSpec((TC_R, D), lambda i: (i, 0)),
            pl.BlockSpec((D, N), lambda i: (0, 0)),
            pl.BlockSpec((TC_R, 1), lambda i: (i, 0)),
            pl.BlockSpec((1, N), lambda i: (0, 0)),
        ],
        out_specs=[
            pl.BlockSpec((TC_R, K), lambda i: (i, 0)),
            pl.BlockSpec((TC_R, K), lambda i: (i, 0)),
        ],
        out_shape=[
            jax.ShapeDtypeStruct((N, K), jnp.float32),
            jax.ShapeDtypeStruct((N, K), jnp.int32),
        ],
    )(x, xT, sq_col, sq_row)


def _sc_combine(idxT, valsT, memflat):
    info = plsc.get_sparse_core_info()
    nw = info.num_cores * info.num_subcores
    rpw = N // nw
    mesh = plsc.VectorSubcoreMesh(core_axis_name="c", subcore_axis_name="s")

    @functools.partial(
        pl.kernel,
        mesh=mesh,
        out_type=jax.ShapeDtypeStruct((N,), jnp.float32),
        scratch_types=[
            pltpu.VMEM((K, rpw), jnp.int32),
            pltpu.VMEM((K, rpw), jnp.int32),
            pltpu.VMEM((K, rpw), jnp.float32),
            pltpu.VMEM((rpw,), jnp.float32),
            pltpu.SemaphoreType.DMA,
        ],
    )
    def k(idx_hbm, vals_hbm, mem_hbm, out_hbm, idx_v, mem_v, vals_v, out_v, sem):
        wid = lax.axis_index("s") * info.num_cores + lax.axis_index("c")
        base = wid * rpw
        nl = info.num_lanes
        pltpu.sync_copy(idx_hbm.at[:, pl.ds(base, rpw)], idx_v)
        pltpu.sync_copy(vals_hbm.at[:, pl.ds(base, rpw)], vals_v)
        for kk in range(K):
            pltpu.async_copy(mem_hbm.at[idx_v.at[kk]], mem_v.at[kk], sem).wait()
        for j in range(rpw // nl):
            s = pl.ds(j * nl, nl)
            acc = vals_v[0, s] * mem_v[0, s].astype(jnp.float32)
            acc = acc + vals_v[1, s] * mem_v[1, s].astype(jnp.float32)
            acc = acc + vals_v[2, s] * mem_v[2, s].astype(jnp.float32)
            out_v[s] = acc / 3.0
        pltpu.sync_copy(out_v, out_hbm.at[pl.ds(base, rpw)])

    return k(idxT, valsT, memflat)


def kernel(items_embeddings, membership):
    x = items_embeddings
    sq = jnp.sum(x * x, axis=1)
    vals, idx = _tc_top3(x, x.T, sq[:, None], sq[None, :])
    return _sc_combine(idx.T, vals.T, membership.reshape(-1))


# speedup vs baseline: 1.6414x; 1.1083x over previous
import functools

import jax
import jax.numpy as jnp
from jax import lax
from jax.experimental import pallas as pl
from jax.experimental.pallas import tpu as pltpu
from jax.experimental.pallas import tpu_sc as plsc

N = 4096
D = 128
TC_R = 256
K = 3


def _tc_body(a_ref, b_ref, sqr_ref, sqc_ref, vals_ref, idx_ref):
    a = a_ref[...]
    b = b_ref[...]
    gram = lax.dot_general(a, b, (((1,), (1,)), ((), ())),
                           preferred_element_type=jnp.float32)
    d2 = sqr_ref[...] + sqc_ref[...] - 2.0 * gram
    big = jnp.float32(jnp.inf)
    work = jnp.where(d2 > 0.0, d2, big)
    row0 = pl.program_id(0) * TC_R
    growsr = row0 + lax.broadcasted_iota(jnp.int32, (1, TC_R), 1)
    colf = lax.broadcasted_iota(jnp.int32, (1, N), 1).astype(jnp.float32)
    bigc = jnp.float32(N)
    for kk in range(K):
        m = jnp.min(work, axis=1, keepdims=True)
        sel = work == m
        idxf = jnp.min(jnp.where(sel, colf, bigc), axis=1, keepdims=True)
        vals_ref[kk:kk + 1, :] = 1.0 / (jnp.sqrt(m.T) + 1.0)
        idx_ref[kk:kk + 1, :] = growsr * N + idxf.T.astype(jnp.int32)
        if kk + 1 < K:
            work = jnp.where(colf == idxf, big, work)


def _tc_top3(x, sq_col, sq_row):
    return pl.pallas_call(
        _tc_body,
        grid=(N // TC_R,),
        in_specs=[
            pl.BlockSpec((TC_R, D), lambda i: (i, 0)),
            pl.BlockSpec((N, D), lambda i: (0, 0)),
            pl.BlockSpec((TC_R, 1), lambda i: (i, 0)),
            pl.BlockSpec((1, N), lambda i: (0, 0)),
        ],
        out_specs=[
            pl.BlockSpec((K, TC_R), lambda i: (0, i)),
            pl.BlockSpec((K, TC_R), lambda i: (0, i)),
        ],
        out_shape=[
            jax.ShapeDtypeStruct((K, N), jnp.float32),
            jax.ShapeDtypeStruct((K, N), jnp.int32),
        ],
    )(x, x, sq_col, sq_row)


def _sc_combine(idxT, valsT, memflat):
    info = plsc.get_sparse_core_info()
    nw = info.num_cores * info.num_subcores
    rpw = N // nw
    mesh = plsc.VectorSubcoreMesh(core_axis_name="c", subcore_axis_name="s")

    @functools.partial(
        pl.kernel,
        mesh=mesh,
        out_type=jax.ShapeDtypeStruct((N,), jnp.float32),
        scratch_types=[
            pltpu.VMEM((K, rpw), jnp.int32),
            pltpu.VMEM((K, rpw), jnp.int32),
            pltpu.VMEM((K, rpw), jnp.float32),
            pltpu.VMEM((rpw,), jnp.float32),
            pltpu.SemaphoreType.DMA,
        ],
    )
    def k(idx_hbm, vals_hbm, mem_hbm, out_hbm, idx_v, mem_v, vals_v, out_v, sem):
        wid = lax.axis_index("s") * info.num_cores + lax.axis_index("c")
        base = wid * rpw
        nl = info.num_lanes
        pltpu.sync_copy(idx_hbm.at[:, pl.ds(base, rpw)], idx_v)
        pltpu.sync_copy(vals_hbm.at[:, pl.ds(base, rpw)], vals_v)
        for kk in range(K):
            pltpu.async_copy(mem_hbm.at[idx_v.at[kk]], mem_v.at[kk], sem).wait()
        for j in range(rpw // nl):
            s = pl.ds(j * nl, nl)
            acc = vals_v[0, s] * mem_v[0, s].astype(jnp.float32)
            acc = acc + vals_v[1, s] * mem_v[1, s].astype(jnp.float32)
            acc = acc + vals_v[2, s] * mem_v[2, s].astype(jnp.float32)
            out_v[s] = acc / 3.0
        pltpu.sync_copy(out_v, out_hbm.at[pl.ds(base, rpw)])

    return k(idxT, valsT, memflat)


def kernel(items_embeddings, membership):
    x = items_embeddings
    sq = jnp.sum(x * x, axis=1)
    vals, idx = _tc_top3(x, sq[:, None], sq[None, :])
    return _sc_combine(idx, vals, membership.reshape(-1))


# speedup vs baseline: 1.6997x; 1.0355x over previous
import functools

import jax
import jax.numpy as jnp
from jax import lax
from jax.experimental import pallas as pl
from jax.experimental.pallas import tpu as pltpu
from jax.experimental.pallas import tpu_sc as plsc

N = 4096
D = 128
TC_R = 512
K = 3


def _tc_body(a_ref, b_ref, sqr_ref, sqc_ref, vals_ref, idx_ref):
    a = a_ref[...]
    b = b_ref[...]
    gram = lax.dot_general(a, b, (((1,), (1,)), ((), ())),
                           preferred_element_type=jnp.float32)
    d2 = sqr_ref[...] + sqc_ref[...] - 2.0 * gram
    big = jnp.float32(jnp.inf)
    work = jnp.where(d2 > 0.0, d2, big)
    row0 = pl.program_id(0) * TC_R
    growsr = row0 + lax.broadcasted_iota(jnp.int32, (1, TC_R), 1)
    colf = lax.broadcasted_iota(jnp.int32, (1, N), 1).astype(jnp.float32)
    bigc = jnp.float32(N)
    for kk in range(K):
        m = jnp.min(work, axis=1, keepdims=True)
        sel = work == m
        idxf = jnp.min(jnp.where(sel, colf, bigc), axis=1, keepdims=True)
        vals_ref[kk:kk + 1, :] = 1.0 / (jnp.sqrt(m.T) + 1.0)
        idx_ref[kk:kk + 1, :] = growsr * N + idxf.T.astype(jnp.int32)
        if kk + 1 < K:
            work = jnp.where(colf == idxf, big, work)


def _tc_top3(x, sq_col, sq_row):
    return pl.pallas_call(
        _tc_body,
        grid=(N // TC_R,),
        in_specs=[
            pl.BlockSpec((TC_R, D), lambda i: (i, 0)),
            pl.BlockSpec((N, D), lambda i: (0, 0)),
            pl.BlockSpec((TC_R, 1), lambda i: (i, 0)),
            pl.BlockSpec((1, N), lambda i: (0, 0)),
        ],
        out_specs=[
            pl.BlockSpec((K, TC_R), lambda i: (0, i)),
            pl.BlockSpec((K, TC_R), lambda i: (0, i)),
        ],
        out_shape=[
            jax.ShapeDtypeStruct((K, N), jnp.float32),
            jax.ShapeDtypeStruct((K, N), jnp.int32),
        ],
    )(x, x, sq_col, sq_row)


def _sc_combine(idxT, valsT, memflat):
    info = plsc.get_sparse_core_info()
    nw = info.num_cores * info.num_subcores
    rpw = N // nw
    mesh = plsc.VectorSubcoreMesh(core_axis_name="c", subcore_axis_name="s")

    @functools.partial(
        pl.kernel,
        mesh=mesh,
        out_type=jax.ShapeDtypeStruct((N,), jnp.float32),
        scratch_types=[
            pltpu.VMEM((K, rpw), jnp.int32),
            pltpu.VMEM((K, rpw), jnp.int32),
            pltpu.VMEM((K, rpw), jnp.float32),
            pltpu.VMEM((rpw,), jnp.float32),
            pltpu.SemaphoreType.DMA,
        ],
    )
    def k(idx_hbm, vals_hbm, mem_hbm, out_hbm, idx_v, mem_v, vals_v, out_v, sem):
        wid = lax.axis_index("s") * info.num_cores + lax.axis_index("c")
        base = wid * rpw
        nl = info.num_lanes
        pltpu.sync_copy(idx_hbm.at[:, pl.ds(base, rpw)], idx_v)
        pltpu.sync_copy(vals_hbm.at[:, pl.ds(base, rpw)], vals_v)
        for kk in range(K):
            pltpu.async_copy(mem_hbm.at[idx_v.at[kk]], mem_v.at[kk], sem).wait()
        for j in range(rpw // nl):
            s = pl.ds(j * nl, nl)
            acc = vals_v[0, s] * mem_v[0, s].astype(jnp.float32)
            acc = acc + vals_v[1, s] * mem_v[1, s].astype(jnp.float32)
            acc = acc + vals_v[2, s] * mem_v[2, s].astype(jnp.float32)
            out_v[s] = acc / 3.0
        pltpu.sync_copy(out_v, out_hbm.at[pl.ds(base, rpw)])

    return k(idxT, valsT, memflat)


def kernel(items_embeddings, membership):
    x = items_embeddings
    sq = jnp.sum(x * x, axis=1)
    vals, idx = _tc_top3(x, sq[:, None], sq[None, :])
    return _sc_combine(idx, vals, membership.reshape(-1))


# speedup vs baseline: 1.7076x; 1.0046x over previous
import functools

import jax
import jax.numpy as jnp
from jax import lax
from jax.experimental import pallas as pl
from jax.experimental.pallas import tpu as pltpu
from jax.experimental.pallas import tpu_sc as plsc

N = 4096
D = 128
TC_R = 1024
K = 3


def _tc_body(a_ref, b_ref, sqr_ref, sqc_ref, vals_ref, idx_ref):
    a = a_ref[...]
    b = b_ref[...]
    gram = lax.dot_general(a, b, (((1,), (1,)), ((), ())),
                           preferred_element_type=jnp.float32)
    d2 = sqr_ref[...] + sqc_ref[...] - 2.0 * gram
    big = jnp.float32(jnp.inf)
    work = jnp.where(d2 > 0.0, d2, big)
    row0 = pl.program_id(0) * TC_R
    growsr = row0 + lax.broadcasted_iota(jnp.int32, (1, TC_R), 1)
    colf = lax.broadcasted_iota(jnp.int32, (1, N), 1).astype(jnp.float32)
    bigc = jnp.float32(N)
    for kk in range(K):
        m = jnp.min(work, axis=1, keepdims=True)
        sel = work == m
        idxf = jnp.min(jnp.where(sel, colf, bigc), axis=1, keepdims=True)
        vals_ref[kk:kk + 1, :] = 1.0 / (jnp.sqrt(m.T) + 1.0)
        idx_ref[kk:kk + 1, :] = growsr * N + idxf.T.astype(jnp.int32)
        if kk + 1 < K:
            work = jnp.where(colf == idxf, big, work)


def _tc_top3(x, sq_col, sq_row):
    return pl.pallas_call(
        _tc_body,
        grid=(N // TC_R,),
        in_specs=[
            pl.BlockSpec((TC_R, D), lambda i: (i, 0)),
            pl.BlockSpec((N, D), lambda i: (0, 0)),
            pl.BlockSpec((TC_R, 1), lambda i: (i, 0)),
            pl.BlockSpec((1, N), lambda i: (0, 0)),
        ],
        out_specs=[
            pl.BlockSpec((K, TC_R), lambda i: (0, i)),
            pl.BlockSpec((K, TC_R), lambda i: (0, i)),
        ],
        out_shape=[
            jax.ShapeDtypeStruct((K, N), jnp.float32),
            jax.ShapeDtypeStruct((K, N), jnp.int32),
        ],
    )(x, x, sq_col, sq_row)


def _sc_combine(idxT, valsT, memflat):
    info = plsc.get_sparse_core_info()
    nw = info.num_cores * info.num_subcores
    rpw = N // nw
    mesh = plsc.VectorSubcoreMesh(core_axis_name="c", subcore_axis_name="s")

    @functools.partial(
        pl.kernel,
        mesh=mesh,
        out_type=jax.ShapeDtypeStruct((N,), jnp.float32),
        scratch_types=[
            pltpu.VMEM((K, rpw), jnp.int32),
            pltpu.VMEM((K, rpw), jnp.int32),
            pltpu.VMEM((K, rpw), jnp.float32),
            pltpu.VMEM((rpw,), jnp.float32),
            pltpu.SemaphoreType.DMA,
        ],
    )
    def k(idx_hbm, vals_hbm, mem_hbm, out_hbm, idx_v, mem_v, vals_v, out_v, sem):
        wid = lax.axis_index("s") * info.num_cores + lax.axis_index("c")
        base = wid * rpw
        nl = info.num_lanes
        pltpu.sync_copy(idx_hbm.at[:, pl.ds(base, rpw)], idx_v)
        pltpu.sync_copy(vals_hbm.at[:, pl.ds(base, rpw)], vals_v)
        for kk in range(K):
            pltpu.async_copy(mem_hbm.at[idx_v.at[kk]], mem_v.at[kk], sem).wait()
        for j in range(rpw // nl):
            s = pl.ds(j * nl, nl)
            acc = vals_v[0, s] * mem_v[0, s].astype(jnp.float32)
            acc = acc + vals_v[1, s] * mem_v[1, s].astype(jnp.float32)
            acc = acc + vals_v[2, s] * mem_v[2, s].astype(jnp.float32)
            out_v[s] = acc / 3.0
        pltpu.sync_copy(out_v, out_hbm.at[pl.ds(base, rpw)])

    return k(idxT, valsT, memflat)


def kernel(items_embeddings, membership):
    x = items_embeddings
    sq = jnp.sum(x * x, axis=1)
    vals, idx = _tc_top3(x, sq[:, None], sq[None, :])
    return _sc_combine(idx, vals, membership.reshape(-1))
